# baseline (speedup 1.0000x reference)
import jax
import jax.numpy as jnp
from jax import lax
from jax.experimental import pallas as pl
from jax.experimental.pallas import tpu as pltpu
from jax.experimental.pallas import tpu_sc as plsc

N = 10000
NP = 10240
E = 320000
EP = 327680
NW = 32
NCHUNK = 80
PAD_NODE = 10200

_f32 = jnp.float32
_i32 = jnp.int32


def _iota16():
    return lax.broadcasted_iota(_i32, (16,), 0)


def _rsqrt16(x):
    i = plsc.bitcast(x, _i32)
    i = jnp.int32(0x5F3759DF) - lax.shift_right_logical(i, 1)
    y = plsc.bitcast(i, _f32)
    for _ in range(3):
        y = y * (jnp.float32(1.5) - jnp.float32(0.5) * x * y * y)
    return y


_MESH = plsc.VectorSubcoreMesh(core_axis_name="c", subcore_axis_name="s")
_CPARAMS = pltpu.CompilerParams(needs_layout_passes=False)


def _k1_body(pred_h, feats_h, src_h, dst_h, col_h, vals_h, row_h, zeros_h,
             deg_o, pg_o, fq_o, agg_o, csr_o,
             pred_v, src_a, dst_a, row_a, cb0, cb1, vv0, vv1,
             fs0, fs1, fd0, fd1, accf,
             dp0, dp1, fb0, fb1, ps0, ps1, pr0, pr1, onesv,
             a_deg, a_pg, a_fq, a_agg, a_csr,
             gs0, gs1, gd0, gd1, sc0, sc1, cs0, cs1):
    cid = lax.axis_index("c")
    sid = lax.axis_index("s")
    wid = cid * 16 + sid
    FS = (fs0, fs1)
    FD = (fd0, fd1)
    CB = (cb0, cb1)
    VV = (vv0, vv1)
    DP = (dp0, dp1)
    FB = (fb0, fb1)
    PS = (ps0, ps1)
    PR = (pr0, pr1)
    GS = (gs0, gs1)
    GD = (gd0, gd1)
    SC = (sc0, sc1)
    CS = (cs0, cs1)

    @pl.when(sid == 0)
    def _():
        pltpu.sync_copy(zeros_h, a_deg)
        pltpu.sync_copy(zeros_h, a_pg)
        pltpu.sync_copy(zeros_h, a_fq)
        pltpu.sync_copy(zeros_h, a_agg)
        pltpu.sync_copy(zeros_h, a_csr)

    pltpu.sync_copy(pred_h, pred_v)
    pltpu.sync_copy(src_h.at[wid], src_a)
    pltpu.sync_copy(dst_h.at[wid], dst_a)
    pltpu.sync_copy(row_h.at[wid], row_a)

    def _fill_ones(i, c):
        onesv[pl.ds(i * 16, 16)] = jnp.ones((16,), _f32)
        return c

    lax.fori_loop(0, 8, _fill_ones, 0)
    plsc.subcore_barrier()

    iota = _iota16()

    def _issue_gather(i, b):
        pltpu.async_copy(feats_h.at[src_a.at[i]], FS[b], GS[b])
        pltpu.async_copy(feats_h.at[dst_a.at[i]], FD[b], GD[b])
        pltpu.async_copy(col_h.at[wid].at[i], CB[b], CS[b])
        pltpu.async_copy(vals_h.at[wid].at[i], VV[b], CS[b])

    def _wait_gather(b):
        pltpu.make_async_copy(feats_h.at[src_a.at[0]], FS[b], GS[b]).wait()
        pltpu.make_async_copy(feats_h.at[dst_a.at[0]], FD[b], GD[b]).wait()
        pltpu.make_async_copy(col_h.at[wid].at[0], CB[b], CS[b]).wait()
        pltpu.make_async_copy(vals_h.at[wid].at[0], VV[b], CS[b]).wait()

    def _issue_scatter(i, b):
        pltpu.async_copy(onesv, a_deg.at[dst_a.at[i]], SC[b], add=True)
        pltpu.async_copy(DP[b], a_pg.at[dst_a.at[i]], SC[b], add=True)
        pltpu.async_copy(FB[b], a_fq.at[dst_a.at[i]], SC[b], add=True)
        pltpu.async_copy(PS[b], a_agg.at[dst_a.at[i]], SC[b], add=True)
        pltpu.async_copy(PR[b], a_csr.at[row_a.at[i]], SC[b], add=True)

    def _wait_scatter(b):
        pltpu.make_async_copy(onesv, a_deg.at[dst_a.at[0]], SC[b]).wait()
        pltpu.make_async_copy(DP[b], a_pg.at[dst_a.at[0]], SC[b]).wait()
        pltpu.make_async_copy(FB[b], a_fq.at[dst_a.at[0]], SC[b]).wait()
        pltpu.make_async_copy(PS[b], a_agg.at[dst_a.at[0]], SC[b]).wait()
        pltpu.make_async_copy(PR[b], a_csr.at[row_a.at[0]], SC[b]).wait()

    def _compute(i, b):
        fsv = FS[b]
        fdv = FD[b]
        for j in range(8):
            off = j * 16
            sidx = src_a[i, pl.ds(off, 16)]
            didx = dst_a[i, pl.ds(off, 16)]
            ps = plsc.load_gather(pred_v, [sidx])
            pd = plsc.load_gather(pred_v, [didx])
            DP[b][pl.ds(off, 16)] = jnp.abs(pd - ps)
            PS[b][pl.ds(off, 16)] = ps
            cidx = CB[b][pl.ds(off, 16)]
            pc = plsc.load_gather(pred_v, [cidx])
            PR[b][pl.ds(off, 16)] = pc * VV[b][pl.ds(off, 16)]
            for e in range(16):
                acc = jnp.zeros((16,), _f32)
                for c in range(4):
                    a = plsc.bitcast(fsv[off + e, pl.ds(c * 16, 16)],
                                     jnp.bfloat16)
                    vb = plsc.bitcast(fdv[off + e, pl.ds(c * 16, 16)],
                                      jnp.bfloat16)
                    d = a - vb
                    lo, hi = plsc.unpack(
                        d, format=plsc.PackFormat.INTERLEAVED)
                    acc = acc + lo * lo + hi * hi
                accf[pl.ds(e * 17, 16)] = acc
            colsum = jnp.zeros((16,), _f32)
            base17 = iota * jnp.int32(17)
            for cc in range(16):
                colsum = colsum + plsc.load_gather(
                    accf, [base17 + jnp.int32(cc)])
            x = colsum + jnp.float32(1e-12)
            FB[b][pl.ds(off, 16)] = x * _rsqrt16(x)

    _issue_gather(0, 0)
    _issue_gather(1, 1)

    def _pair(p, carry):
        for b in range(2):
            i = p * 2 + b
            _wait_gather(b)
            pl.when(i >= 2)(lambda b=b: _wait_scatter(b))
            _compute(i, b)
            pl.when(i + 2 < NCHUNK)(lambda i=i, b=b: _issue_gather(i + 2, b))
            _issue_scatter(i, b)
        return carry

    lax.fori_loop(0, NCHUNK // 2, _pair, 0)
    _wait_scatter(0)
    _wait_scatter(1)
    plsc.subcore_barrier()

    @pl.when(sid == 0)
    def _():
        pltpu.sync_copy(a_deg, deg_o.at[cid])
        pltpu.sync_copy(a_pg, pg_o.at[cid])
        pltpu.sync_copy(a_fq, fq_o.at[cid])
        pltpu.sync_copy(a_agg, agg_o.at[cid])
        pltpu.sync_copy(a_csr, csr_o.at[cid])


def _k2_body(deg_p, pg_p, fq_p, src_h, dst_h, zeros_h,
             s1_o, w_o, degc_o, draw_o,
             d0, d1, p0, p1, f0, f1, degc_v, draw_v, w_v,
             src_a, dst_a, valb, a_s1):
    cid = lax.axis_index("c")
    sid = lax.axis_index("s")
    wid = cid * 16 + sid

    @pl.when(sid == 0)
    def _():
        pltpu.sync_copy(zeros_h, a_s1)

    pltpu.sync_copy(deg_p.at[0], d0)
    pltpu.sync_copy(deg_p.at[1], d1)
    pltpu.sync_copy(pg_p.at[0], p0)
    pltpu.sync_copy(pg_p.at[1], p1)
    pltpu.sync_copy(fq_p.at[0], f0)
    pltpu.sync_copy(fq_p.at[1], f1)
    pltpu.sync_copy(src_h.at[wid], src_a)
    pltpu.sync_copy(dst_h.at[wid], dst_a)

    iota = _iota16()
    one = jnp.float32(1.0)

    def _pass1(i, carry):
        apg, afq = carry
        sl = pl.ds(i * 16, 16)
        deg = d0[sl] + d1[sl]
        degc = jnp.maximum(deg, one)
        draw_v[sl] = deg
        degc_v[sl] = degc
        pg = (p0[sl] + p1[sl]) / degc
        fq = (f0[sl] + f1[sl]) / degc
        return (apg + pg, afq + fq)

    apg, afq = lax.fori_loop(0, NP // 16, _pass1,
                             (jnp.zeros((16,), _f32), jnp.zeros((16,), _f32)))
    inv_n = jnp.float32(1.0 / N)
    mpg = jnp.sum(apg) * inv_n + jnp.float32(1e-8)
    mfq = jnp.sum(afq) * inv_n + jnp.float32(1e-8)
    onev = jnp.full((16,), one, _f32)
    impg = onev / jnp.full((16,), mpg, _f32)
    imfq = onev / jnp.full((16,), mfq, _f32)

    def _pass2(i, carry):
        sl = pl.ds(i * 16, 16)
        degc = degc_v[sl]
        pg = (p0[sl] + p1[sl]) / degc
        fq = (f0[sl] + f1[sl]) / degc
        w = (one + pg * impg) * (one + fq * imfq)
        nid = i * 16 + iota
        w_v[sl] = jnp.where(nid < N, w, jnp.float32(0.0))
        return carry

    lax.fori_loop(0, NP // 16, _pass2, 0)

    @pl.when(jnp.logical_and(cid == 0, sid == 0))
    def _():
        pltpu.sync_copy(w_v, w_o)
        pltpu.sync_copy(degc_v, degc_o)
        pltpu.sync_copy(draw_v, draw_o)

    plsc.subcore_barrier()

    def _chunk(i, carry):
        for j in range(8):
            off = j * 16
            valb[pl.ds(off, 16)] = plsc.load_gather(
                w_v, [src_a[i, pl.ds(off, 16)]])
        pltpu.sync_copy(valb, a_s1.at[dst_a.at[i]], add=True)
        return carry

    lax.fori_loop(0, NCHUNK, _chunk, 0)
    plsc.subcore_barrier()

    @pl.when(sid == 0)
    def _():
        pltpu.sync_copy(a_s1, s1_o.at[cid])


def _k3_body(s1_p, degc_h, src_h, dst_h, zeros_h,
             s2_o, c1_o,
             s10, s11, degc_v, c1_v, src_a, dst_a, valb, a_s2):
    cid = lax.axis_index("c")
    sid = lax.axis_index("s")
    wid = cid * 16 + sid

    @pl.when(sid == 0)
    def _():
        pltpu.sync_copy(zeros_h, a_s2)

    pltpu.sync_copy(s1_p.at[0], s10)
    pltpu.sync_copy(s1_p.at[1], s11)
    pltpu.sync_copy(degc_h, degc_v)
    pltpu.sync_copy(src_h.at[wid], src_a)
    pltpu.sync_copy(dst_h.at[wid], dst_a)

    def _node(i, carry):
        sl = pl.ds(i * 16, 16)
        c1_v[sl] = (s10[sl] + s11[sl]) / degc_v[sl]
        return carry

    lax.fori_loop(0, NP // 16, _node, 0)

    @pl.when(jnp.logical_and(cid == 0, sid == 0))
    def _():
        pltpu.sync_copy(c1_v, c1_o)

    plsc.subcore_barrier()

    def _chunk(i, carry):
        for j in range(8):
            off = j * 16
            valb[pl.ds(off, 16)] = plsc.load_gather(
                c1_v, [src_a[i, pl.ds(off, 16)]])
        pltpu.sync_copy(valb, a_s2.at[dst_a.at[i]], add=True)
        return carry

    lax.fori_loop(0, NCHUNK, _chunk, 0)
    plsc.subcore_barrier()

    @pl.when(sid == 0)
    def _():
        pltpu.sync_copy(a_s2, s2_o.at[cid])


def _k4_body(w_h, degc_h, draw_h, c1_h, s2_p, csr_p, agg_p, pred_h, b_h,
             out_o,
             w_v, degc_v, draw_v, c1_v, s20, s21, cs0, cs1, ag0, ag1,
             pred_v, b_v, outst):
    cid = lax.axis_index("c")
    sid = lax.axis_index("s")

    pltpu.sync_copy(w_h, w_v)
    pltpu.sync_copy(degc_h, degc_v)
    pltpu.sync_copy(draw_h, draw_v)
    pltpu.sync_copy(c1_h, c1_v)
    pltpu.sync_copy(s2_p.at[0], s20)
    pltpu.sync_copy(s2_p.at[1], s21)
    pltpu.sync_copy(csr_p.at[0], cs0)
    pltpu.sync_copy(csr_p.at[1], cs1)
    pltpu.sync_copy(agg_p.at[0], ag0)
    pltpu.sync_copy(agg_p.at[1], ag1)
    pltpu.sync_copy(pred_h, pred_v)
    pltpu.sync_copy(b_h, b_v)

    def _node(i, carry):
        a_w, a_out, a_or, a_dv = carry
        sl = pl.ds(i * 16, 16)
        w = w_v[sl]
        c1 = c1_v[sl]
        c2 = (s20[sl] + s21[sl]) / degc_v[sl]
        out_u = w + jnp.float32(0.5) * c1 + jnp.float32(0.25) * c2
        r = cs0[sl] + cs1[sl] - b_v[sl]
        div = draw_v[sl] * pred_v[sl] - (ag0[sl] + ag1[sl])
        return (a_w + w, a_out + out_u, a_or + out_u * r * r,
                a_dv + div * div)

    z = jnp.zeros((16,), _f32)
    a_w, a_out, a_or, a_dv = lax.fori_loop(0, NP // 16, _node, (z, z, z, z))
    inv_n = jnp.float32(1.0 / N)
    sw = jnp.sum(a_w)
    sout = jnp.sum(a_out)
    sor = jnp.sum(a_or)
    sdv = jnp.sum(a_dv)
    mw = sw * inv_n + jnp.float32(1e-8)
    denom = sout * inv_n + jnp.float32(1e-8) * mw
    loss_pde_v = (jnp.full((16,), sor * inv_n, _f32)
                  / jnp.full((16,), denom, _f32))
    loss_v = loss_pde_v + jnp.float32(0.1) * (sdv * inv_n)
    outst[:] = jnp.where(_iota16() == 0, loss_v, jnp.float32(0.0))

    @pl.when(jnp.logical_and(cid == 0, sid == 0))
    def _():
        pltpu.sync_copy(outst, out_o)


def _sds(shape, dtype=_f32):
    return jax.ShapeDtypeStruct(shape, dtype)


_k1 = pl.kernel(
    _k1_body,
    out_type=[_sds((2, NP))] * 5,
    mesh=_MESH,
    compiler_params=_CPARAMS,
    scratch_types=[
        pltpu.VMEM((NP,), _f32),
        pltpu.VMEM((NCHUNK, 128), _i32),
        pltpu.VMEM((NCHUNK, 128), _i32),
        pltpu.VMEM((NCHUNK, 128), _i32),
        pltpu.VMEM((128,), _i32),
        pltpu.VMEM((128,), _i32),
        pltpu.VMEM((128,), _f32),
        pltpu.VMEM((128,), _f32),
        pltpu.VMEM((128, 128), _i32),
        pltpu.VMEM((128, 128), _i32),
        pltpu.VMEM((128, 128), _i32),
        pltpu.VMEM((128, 128), _i32),
        pltpu.VMEM((272,), _f32),
        pltpu.VMEM((128,), _f32),
        pltpu.VMEM((128,), _f32),
        pltpu.VMEM((128,), _f32),
        pltpu.VMEM((128,), _f32),
        pltpu.VMEM((128,), _f32),
        pltpu.VMEM((128,), _f32),
        pltpu.VMEM((128,), _f32),
        pltpu.VMEM((128,), _f32),
        pltpu.VMEM((128,), _f32),
        pltpu.VMEM_SHARED((NP,), _f32),
        pltpu.VMEM_SHARED((NP,), _f32),
        pltpu.VMEM_SHARED((NP,), _f32),
        pltpu.VMEM_SHARED((NP,), _f32),
        pltpu.VMEM_SHARED((NP,), _f32),
        pltpu.SemaphoreType.DMA,
        pltpu.SemaphoreType.DMA,
        pltpu.SemaphoreType.DMA,
        pltpu.SemaphoreType.DMA,
        pltpu.SemaphoreType.DMA,
        pltpu.SemaphoreType.DMA,
        pltpu.SemaphoreType.DMA,
        pltpu.SemaphoreType.DMA,
    ],
)

_k2 = pl.kernel(
    _k2_body,
    out_type=[_sds((2, NP)), _sds((NP,)), _sds((NP,)), _sds((NP,))],
    mesh=_MESH,
    compiler_params=_CPARAMS,
    scratch_types=[
        pltpu.VMEM((NP,), _f32),
        pltpu.VMEM((NP,), _f32),
        pltpu.VMEM((NP,), _f32),
        pltpu.VMEM((NP,), _f32),
        pltpu.VMEM((NP,), _f32),
        pltpu.VMEM((NP,), _f32),
        pltpu.VMEM((NP,), _f32),
        pltpu.VMEM((NP,), _f32),
        pltpu.VMEM((NP,), _f32),
        pltpu.VMEM((NCHUNK, 128), _i32),
        pltpu.VMEM((NCHUNK, 128), _i32),
        pltpu.VMEM((128,), _f32),
        pltpu.VMEM_SHARED((NP,), _f32),
    ],
)

_k3 = pl.kernel(
    _k3_body,
    out_type=[_sds((2, NP)), _sds((NP,))],
    mesh=_MESH,
    compiler_params=_CPARAMS,
    scratch_types=[
        pltpu.VMEM((NP,), _f32),
        pltpu.VMEM((NP,), _f32),
        pltpu.VMEM((NP,), _f32),
        pltpu.VMEM((NP,), _f32),
        pltpu.VMEM((NCHUNK, 128), _i32),
        pltpu.VMEM((NCHUNK, 128), _i32),
        pltpu.VMEM((128,), _f32),
        pltpu.VMEM_SHARED((NP,), _f32),
    ],
)

_k4 = pl.kernel(
    _k4_body,
    out_type=[_sds((16,))],
    mesh=_MESH,
    compiler_params=_CPARAMS,
    scratch_types=[
        pltpu.VMEM((NP,), _f32),
        pltpu.VMEM((NP,), _f32),
        pltpu.VMEM((NP,), _f32),
        pltpu.VMEM((NP,), _f32),
        pltpu.VMEM((NP,), _f32),
        pltpu.VMEM((NP,), _f32),
        pltpu.VMEM((NP,), _f32),
        pltpu.VMEM((NP,), _f32),
        pltpu.VMEM((NP,), _f32),
        pltpu.VMEM((NP,), _f32),
        pltpu.VMEM((NP,), _f32),
        pltpu.VMEM((NP,), _f32),
        pltpu.VMEM((16,), _f32),
    ],
)


def kernel(pred, feats, row_ptr, col_ind, vals, row_idx, b, edge_index, epoch):
    pred_p = jnp.pad(pred.astype(_f32), (0, NP - N))
    b_p = jnp.pad(b.astype(_f32), (0, NP - N))
    fbits = lax.bitcast_convert_type(
        jnp.pad(feats.astype(jnp.bfloat16),
                ((0, NP - N), (0, 0))).reshape(NP, 64, 2),
        _i32)
    feats_p = jnp.concatenate([fbits, fbits], axis=1)
    src = jnp.pad(edge_index[0].astype(_i32), (0, EP - E),
                  constant_values=PAD_NODE).reshape(NW, NCHUNK, 128)
    dst = jnp.pad(edge_index[1].astype(_i32), (0, EP - E),
                  constant_values=PAD_NODE).reshape(NW, NCHUNK, 128)
    col = jnp.pad(col_ind.astype(_i32), (0, EP - E),
                  constant_values=PAD_NODE).reshape(NW, NCHUNK, 128)
    row = jnp.pad(row_idx.astype(_i32), (0, EP - E),
                  constant_values=PAD_NODE).reshape(NW, NCHUNK, 128)
    vls = jnp.pad(vals.astype(_f32), (0, EP - E)).reshape(NW, NCHUNK, 128)
    zeros = jnp.zeros((NP,), _f32)

    deg_p, pg_p, fq_p, agg_p, csr_p = _k1(
        pred_p, feats_p, src, dst, col, vls, row, zeros)
    s1_p, w_arr, degc, draw = _k2(deg_p, pg_p, fq_p, src, dst, zeros)
    s2_p, c1 = _k3(s1_p, degc, src, dst, zeros)
    (out16,) = _k4(w_arr, degc, draw, c1, s2_p, csr_p, agg_p, pred_p, b_p)
    return out16[0]


# speedup vs baseline: 1.1348x; 1.1348x over previous
import jax
import jax.numpy as jnp
from jax import lax
from jax.experimental import pallas as pl
from jax.experimental.pallas import tpu as pltpu
from jax.experimental.pallas import tpu_sc as plsc

N = 10000
NP = 10240
E = 320000
EP = 327680
NW = 32
NCHUNK = 80
PAD_NODE = 10200

_f32 = jnp.float32
_i32 = jnp.int32


def _iota16():
    return lax.broadcasted_iota(_i32, (16,), 0)


def _rsqrt16(x):
    i = plsc.bitcast(x, _i32)
    i = jnp.int32(0x5F3759DF) - lax.shift_right_logical(i, 1)
    y = plsc.bitcast(i, _f32)
    for _ in range(3):
        y = y * (jnp.float32(1.5) - jnp.float32(0.5) * x * y * y)
    return y


_MESH = plsc.VectorSubcoreMesh(core_axis_name="c", subcore_axis_name="s")
_CPARAMS = pltpu.CompilerParams(needs_layout_passes=False)


def _k1_body(pred_h, feats_h, src_h, dst_h, col_h, vals_h, row_h, zeros_h,
             deg_o, pg_o, fq_o, agg_o, csr_o,
             pred_v, src_a, dst_a, row_a, cb0, cb1, vv0, vv1,
             fs0, fs1, fd0, fd1, accf,
             dp0, dp1, fb0, fb1, ps0, ps1, pr0, pr1, onesv,
             a_deg, a_pg, a_fq, a_agg, a_csr,
             gs0, gs1, gd0, gd1, sc0, sc1, cs0, cs1):
    cid = lax.axis_index("c")
    sid = lax.axis_index("s")
    wid = cid * 16 + sid
    FS = (fs0, fs1)
    FD = (fd0, fd1)
    CB = (cb0, cb1)
    VV = (vv0, vv1)
    DP = (dp0, dp1)
    FB = (fb0, fb1)
    PS = (ps0, ps1)
    PR = (pr0, pr1)
    GS = (gs0, gs1)
    GD = (gd0, gd1)
    SC = (sc0, sc1)
    CS = (cs0, cs1)

    @pl.when(sid == 0)
    def _():
        pltpu.sync_copy(zeros_h, a_deg)
        pltpu.sync_copy(zeros_h, a_pg)
        pltpu.sync_copy(zeros_h, a_fq)
        pltpu.sync_copy(zeros_h, a_agg)
        pltpu.sync_copy(zeros_h, a_csr)

    pltpu.sync_copy(pred_h, pred_v)
    pltpu.sync_copy(src_h.at[wid], src_a)
    pltpu.sync_copy(dst_h.at[wid], dst_a)
    pltpu.sync_copy(row_h.at[wid], row_a)

    def _fill_ones(i, c):
        onesv[pl.ds(i * 16, 16)] = jnp.ones((16,), _f32)
        return c

    lax.fori_loop(0, 8, _fill_ones, 0)
    plsc.subcore_barrier()

    iota = _iota16()

    def _issue_gather(i, b):
        pltpu.async_copy(feats_h.at[src_a.at[i]], FS[b], GS[b])
        pltpu.async_copy(feats_h.at[dst_a.at[i]], FD[b], GD[b])
        pltpu.async_copy(col_h.at[wid].at[i], CB[b], CS[b])
        pltpu.async_copy(vals_h.at[wid].at[i], VV[b], CS[b])

    def _wait_gather(b):
        pltpu.make_async_copy(feats_h.at[src_a.at[0]], FS[b], GS[b]).wait()
        pltpu.make_async_copy(feats_h.at[dst_a.at[0]], FD[b], GD[b]).wait()
        pltpu.make_async_copy(col_h.at[wid].at[0], CB[b], CS[b]).wait()
        pltpu.make_async_copy(vals_h.at[wid].at[0], VV[b], CS[b]).wait()

    def _issue_scatter(i, b):
        pltpu.async_copy(onesv, a_deg.at[dst_a.at[i]], SC[b], add=True)
        pltpu.async_copy(DP[b], a_pg.at[dst_a.at[i]], SC[b], add=True)
        pltpu.async_copy(FB[b], a_fq.at[dst_a.at[i]], SC[b], add=True)
        pltpu.async_copy(PS[b], a_agg.at[dst_a.at[i]], SC[b], add=True)
        pltpu.async_copy(PR[b], a_csr.at[row_a.at[i]], SC[b], add=True)

    def _wait_scatter(b):
        pltpu.make_async_copy(onesv, a_deg.at[dst_a.at[0]], SC[b]).wait()
        pltpu.make_async_copy(DP[b], a_pg.at[dst_a.at[0]], SC[b]).wait()
        pltpu.make_async_copy(FB[b], a_fq.at[dst_a.at[0]], SC[b]).wait()
        pltpu.make_async_copy(PS[b], a_agg.at[dst_a.at[0]], SC[b]).wait()
        pltpu.make_async_copy(PR[b], a_csr.at[row_a.at[0]], SC[b]).wait()

    def _compute(i, b):
        fsv = FS[b]
        fdv = FD[b]
        for j in range(8):
            off = j * 16
            sidx = src_a[i, pl.ds(off, 16)]
            didx = dst_a[i, pl.ds(off, 16)]
            ps = plsc.load_gather(pred_v, [sidx])
            pd = plsc.load_gather(pred_v, [didx])
            DP[b][pl.ds(off, 16)] = jnp.abs(pd - ps)
            PS[b][pl.ds(off, 16)] = ps
            cidx = CB[b][pl.ds(off, 16)]
            pc = plsc.load_gather(pred_v, [cidx])
            PR[b][pl.ds(off, 16)] = pc * VV[b][pl.ds(off, 16)]
            for e in range(16):
                acc = jnp.zeros((16,), _f32)
                for c in range(8):
                    va = fsv[off + e, pl.ds(c * 16, 16)]
                    vb = fdv[off + e, pl.ds(c * 16, 16)]
                    d = va - vb
                    acc = acc + d * d
                accf[pl.ds(e * 17, 16)] = acc
            colsum = jnp.zeros((16,), _f32)
            base17 = iota * jnp.int32(17)
            for cc in range(16):
                colsum = colsum + plsc.load_gather(
                    accf, [base17 + jnp.int32(cc)])
            x = colsum + jnp.float32(1e-12)
            FB[b][pl.ds(off, 16)] = x * _rsqrt16(x)

    _issue_gather(0, 0)
    _issue_gather(1, 1)

    def _pair(p, carry):
        for b in range(2):
            i = p * 2 + b
            _wait_gather(b)
            pl.when(i >= 2)(lambda b=b: _wait_scatter(b))
            _compute(i, b)
            pl.when(i + 2 < NCHUNK)(lambda i=i, b=b: _issue_gather(i + 2, b))
            _issue_scatter(i, b)
        return carry

    lax.fori_loop(0, NCHUNK // 2, _pair, 0)
    _wait_scatter(0)
    _wait_scatter(1)
    plsc.subcore_barrier()

    @pl.when(sid == 0)
    def _():
        pltpu.sync_copy(a_deg, deg_o.at[cid])
        pltpu.sync_copy(a_pg, pg_o.at[cid])
        pltpu.sync_copy(a_fq, fq_o.at[cid])
        pltpu.sync_copy(a_agg, agg_o.at[cid])
        pltpu.sync_copy(a_csr, csr_o.at[cid])


def _k2_body(deg_p, pg_p, fq_p, src_h, dst_h, zeros_h,
             s1_o, w_o, degc_o, draw_o,
             d0, d1, p0, p1, f0, f1, degc_v, draw_v, w_v,
             src_a, dst_a, vb0, vb1, a_s1, sb0, sb1):
    cid = lax.axis_index("c")
    sid = lax.axis_index("s")
    wid = cid * 16 + sid

    @pl.when(sid == 0)
    def _():
        pltpu.sync_copy(zeros_h, a_s1)

    pltpu.sync_copy(deg_p.at[0], d0)
    pltpu.sync_copy(deg_p.at[1], d1)
    pltpu.sync_copy(pg_p.at[0], p0)
    pltpu.sync_copy(pg_p.at[1], p1)
    pltpu.sync_copy(fq_p.at[0], f0)
    pltpu.sync_copy(fq_p.at[1], f1)
    pltpu.sync_copy(src_h.at[wid], src_a)
    pltpu.sync_copy(dst_h.at[wid], dst_a)

    iota = _iota16()
    one = jnp.float32(1.0)

    def _pass1(i, carry):
        apg, afq = carry
        sl = pl.ds(i * 16, 16)
        deg = d0[sl] + d1[sl]
        degc = jnp.maximum(deg, one)
        draw_v[sl] = deg
        degc_v[sl] = degc
        pg = (p0[sl] + p1[sl]) / degc
        fq = (f0[sl] + f1[sl]) / degc
        return (apg + pg, afq + fq)

    apg, afq = lax.fori_loop(0, NP // 16, _pass1,
                             (jnp.zeros((16,), _f32), jnp.zeros((16,), _f32)))
    inv_n = jnp.float32(1.0 / N)
    mpg = jnp.sum(apg) * inv_n + jnp.float32(1e-8)
    mfq = jnp.sum(afq) * inv_n + jnp.float32(1e-8)
    onev = jnp.full((16,), one, _f32)
    impg = onev / jnp.full((16,), mpg, _f32)
    imfq = onev / jnp.full((16,), mfq, _f32)

    def _pass2(i, carry):
        sl = pl.ds(i * 16, 16)
        degc = degc_v[sl]
        pg = (p0[sl] + p1[sl]) / degc
        fq = (f0[sl] + f1[sl]) / degc
        w = (one + pg * impg) * (one + fq * imfq)
        nid = i * 16 + iota
        w_v[sl] = jnp.where(nid < N, w, jnp.float32(0.0))
        return carry

    lax.fori_loop(0, NP // 16, _pass2, 0)

    @pl.when(jnp.logical_and(cid == 0, sid == 0))
    def _():
        pltpu.sync_copy(w_v, w_o)
        pltpu.sync_copy(degc_v, degc_o)
        pltpu.sync_copy(draw_v, draw_o)

    plsc.subcore_barrier()

    VB = (vb0, vb1)
    SB = (sb0, sb1)

    def _pair(p, carry):
        for b in range(2):
            i = p * 2 + b
            pl.when(i >= 2)(lambda b=b: pltpu.make_async_copy(
                VB[b], a_s1.at[dst_a.at[0]], SB[b]).wait())
            for j in range(8):
                off = j * 16
                VB[b][pl.ds(off, 16)] = plsc.load_gather(
                    w_v, [src_a[i, pl.ds(off, 16)]])
            pltpu.async_copy(VB[b], a_s1.at[dst_a.at[i]], SB[b], add=True)
        return carry

    lax.fori_loop(0, NCHUNK // 2, _pair, 0)
    pltpu.make_async_copy(vb0, a_s1.at[dst_a.at[0]], sb0).wait()
    pltpu.make_async_copy(vb1, a_s1.at[dst_a.at[0]], sb1).wait()
    plsc.subcore_barrier()

    @pl.when(sid == 0)
    def _():
        pltpu.sync_copy(a_s1, s1_o.at[cid])


def _k3_body(s1_p, degc_h, src_h, dst_h, zeros_h,
             s2_o, c1_o,
             s10, s11, degc_v, c1_v, src_a, dst_a, vb0, vb1, a_s2,
             sb0, sb1):
    cid = lax.axis_index("c")
    sid = lax.axis_index("s")
    wid = cid * 16 + sid

    @pl.when(sid == 0)
    def _():
        pltpu.sync_copy(zeros_h, a_s2)

    pltpu.sync_copy(s1_p.at[0], s10)
    pltpu.sync_copy(s1_p.at[1], s11)
    pltpu.sync_copy(degc_h, degc_v)
    pltpu.sync_copy(src_h.at[wid], src_a)
    pltpu.sync_copy(dst_h.at[wid], dst_a)

    def _node(i, carry):
        sl = pl.ds(i * 16, 16)
        c1_v[sl] = (s10[sl] + s11[sl]) / degc_v[sl]
        return carry

    lax.fori_loop(0, NP // 16, _node, 0)

    @pl.when(jnp.logical_and(cid == 0, sid == 0))
    def _():
        pltpu.sync_copy(c1_v, c1_o)

    plsc.subcore_barrier()

    VB = (vb0, vb1)
    SB = (sb0, sb1)

    def _pair(p, carry):
        for b in range(2):
            i = p * 2 + b
            pl.when(i >= 2)(lambda b=b: pltpu.make_async_copy(
                VB[b], a_s2.at[dst_a.at[0]], SB[b]).wait())
            for j in range(8):
                off = j * 16
                VB[b][pl.ds(off, 16)] = plsc.load_gather(
                    c1_v, [src_a[i, pl.ds(off, 16)]])
            pltpu.async_copy(VB[b], a_s2.at[dst_a.at[i]], SB[b], add=True)
        return carry

    lax.fori_loop(0, NCHUNK // 2, _pair, 0)
    pltpu.make_async_copy(vb0, a_s2.at[dst_a.at[0]], sb0).wait()
    pltpu.make_async_copy(vb1, a_s2.at[dst_a.at[0]], sb1).wait()
    plsc.subcore_barrier()

    @pl.when(sid == 0)
    def _():
        pltpu.sync_copy(a_s2, s2_o.at[cid])


def _k4_body(w_h, degc_h, draw_h, c1_h, s2_p, csr_p, agg_p, pred_h, b_h,
             out_o,
             w_v, degc_v, draw_v, c1_v, s20, s21, cs0, cs1, ag0, ag1,
             pred_v, b_v, outst):
    cid = lax.axis_index("c")
    sid = lax.axis_index("s")

    pltpu.sync_copy(w_h, w_v)
    pltpu.sync_copy(degc_h, degc_v)
    pltpu.sync_copy(draw_h, draw_v)
    pltpu.sync_copy(c1_h, c1_v)
    pltpu.sync_copy(s2_p.at[0], s20)
    pltpu.sync_copy(s2_p.at[1], s21)
    pltpu.sync_copy(csr_p.at[0], cs0)
    pltpu.sync_copy(csr_p.at[1], cs1)
    pltpu.sync_copy(agg_p.at[0], ag0)
    pltpu.sync_copy(agg_p.at[1], ag1)
    pltpu.sync_copy(pred_h, pred_v)
    pltpu.sync_copy(b_h, b_v)

    def _node(i, carry):
        a_w, a_out, a_or, a_dv = carry
        sl = pl.ds(i * 16, 16)
        w = w_v[sl]
        c1 = c1_v[sl]
        c2 = (s20[sl] + s21[sl]) / degc_v[sl]
        out_u = w + jnp.float32(0.5) * c1 + jnp.float32(0.25) * c2
        r = cs0[sl] + cs1[sl] - b_v[sl]
        div = draw_v[sl] * pred_v[sl] - (ag0[sl] + ag1[sl])
        return (a_w + w, a_out + out_u, a_or + out_u * r * r,
                a_dv + div * div)

    z = jnp.zeros((16,), _f32)
    a_w, a_out, a_or, a_dv = lax.fori_loop(0, NP // 16, _node, (z, z, z, z))
    inv_n = jnp.float32(1.0 / N)
    sw = jnp.sum(a_w)
    sout = jnp.sum(a_out)
    sor = jnp.sum(a_or)
    sdv = jnp.sum(a_dv)
    mw = sw * inv_n + jnp.float32(1e-8)
    denom = sout * inv_n + jnp.float32(1e-8) * mw
    loss_pde_v = (jnp.full((16,), sor * inv_n, _f32)
                  / jnp.full((16,), denom, _f32))
    loss_v = loss_pde_v + jnp.float32(0.1) * (sdv * inv_n)
    outst[:] = jnp.where(_iota16() == 0, loss_v, jnp.float32(0.0))

    @pl.when(jnp.logical_and(cid == 0, sid == 0))
    def _():
        pltpu.sync_copy(outst, out_o)


def _sds(shape, dtype=_f32):
    return jax.ShapeDtypeStruct(shape, dtype)


_k1 = pl.kernel(
    _k1_body,
    out_type=[_sds((2, NP))] * 5,
    mesh=_MESH,
    compiler_params=_CPARAMS,
    scratch_types=[
        pltpu.VMEM((NP,), _f32),
        pltpu.VMEM((NCHUNK, 128), _i32),
        pltpu.VMEM((NCHUNK, 128), _i32),
        pltpu.VMEM((NCHUNK, 128), _i32),
        pltpu.VMEM((128,), _i32),
        pltpu.VMEM((128,), _i32),
        pltpu.VMEM((128,), _f32),
        pltpu.VMEM((128,), _f32),
        pltpu.VMEM((128, 128), _f32),
        pltpu.VMEM((128, 128), _f32),
        pltpu.VMEM((128, 128), _f32),
        pltpu.VMEM((128, 128), _f32),
        pltpu.VMEM((272,), _f32),
        pltpu.VMEM((128,), _f32),
        pltpu.VMEM((128,), _f32),
        pltpu.VMEM((128,), _f32),
        pltpu.VMEM((128,), _f32),
        pltpu.VMEM((128,), _f32),
        pltpu.VMEM((128,), _f32),
        pltpu.VMEM((128,), _f32),
        pltpu.VMEM((128,), _f32),
        pltpu.VMEM((128,), _f32),
        pltpu.VMEM_SHARED((NP,), _f32),
        pltpu.VMEM_SHARED((NP,), _f32),
        pltpu.VMEM_SHARED((NP,), _f32),
        pltpu.VMEM_SHARED((NP,), _f32),
        pltpu.VMEM_SHARED((NP,), _f32),
        pltpu.SemaphoreType.DMA,
        pltpu.SemaphoreType.DMA,
        pltpu.SemaphoreType.DMA,
        pltpu.SemaphoreType.DMA,
        pltpu.SemaphoreType.DMA,
        pltpu.SemaphoreType.DMA,
        pltpu.SemaphoreType.DMA,
        pltpu.SemaphoreType.DMA,
    ],
)

_k2 = pl.kernel(
    _k2_body,
    out_type=[_sds((2, NP)), _sds((NP,)), _sds((NP,)), _sds((NP,))],
    mesh=_MESH,
    compiler_params=_CPARAMS,
    scratch_types=[
        pltpu.VMEM((NP,), _f32),
        pltpu.VMEM((NP,), _f32),
        pltpu.VMEM((NP,), _f32),
        pltpu.VMEM((NP,), _f32),
        pltpu.VMEM((NP,), _f32),
        pltpu.VMEM((NP,), _f32),
        pltpu.VMEM((NP,), _f32),
        pltpu.VMEM((NP,), _f32),
        pltpu.VMEM((NP,), _f32),
        pltpu.VMEM((NCHUNK, 128), _i32),
        pltpu.VMEM((NCHUNK, 128), _i32),
        pltpu.VMEM((128,), _f32),
        pltpu.VMEM((128,), _f32),
        pltpu.VMEM_SHARED((NP,), _f32),
        pltpu.SemaphoreType.DMA,
        pltpu.SemaphoreType.DMA,
    ],
)

_k3 = pl.kernel(
    _k3_body,
    out_type=[_sds((2, NP)), _sds((NP,))],
    mesh=_MESH,
    compiler_params=_CPARAMS,
    scratch_types=[
        pltpu.VMEM((NP,), _f32),
        pltpu.VMEM((NP,), _f32),
        pltpu.VMEM((NP,), _f32),
        pltpu.VMEM((NP,), _f32),
        pltpu.VMEM((NCHUNK, 128), _i32),
        pltpu.VMEM((NCHUNK, 128), _i32),
        pltpu.VMEM((128,), _f32),
        pltpu.VMEM((128,), _f32),
        pltpu.VMEM_SHARED((NP,), _f32),
        pltpu.SemaphoreType.DMA,
        pltpu.SemaphoreType.DMA,
    ],
)

_k4 = pl.kernel(
    _k4_body,
    out_type=[_sds((16,))],
    mesh=_MESH,
    compiler_params=_CPARAMS,
    scratch_types=[
        pltpu.VMEM((NP,), _f32),
        pltpu.VMEM((NP,), _f32),
        pltpu.VMEM((NP,), _f32),
        pltpu.VMEM((NP,), _f32),
        pltpu.VMEM((NP,), _f32),
        pltpu.VMEM((NP,), _f32),
        pltpu.VMEM((NP,), _f32),
        pltpu.VMEM((NP,), _f32),
        pltpu.VMEM((NP,), _f32),
        pltpu.VMEM((NP,), _f32),
        pltpu.VMEM((NP,), _f32),
        pltpu.VMEM((NP,), _f32),
        pltpu.VMEM((16,), _f32),
    ],
)


def kernel(pred, feats, row_ptr, col_ind, vals, row_idx, b, edge_index, epoch):
    pred_p = jnp.pad(pred.astype(_f32), (0, NP - N))
    b_p = jnp.pad(b.astype(_f32), (0, NP - N))
    feats_p = jnp.pad(feats.astype(_f32), ((0, NP - N), (0, 0)))
    src = jnp.pad(edge_index[0].astype(_i32), (0, EP - E),
                  constant_values=PAD_NODE).reshape(NW, NCHUNK, 128)
    dst = jnp.pad(edge_index[1].astype(_i32), (0, EP - E),
                  constant_values=PAD_NODE).reshape(NW, NCHUNK, 128)
    col = jnp.pad(col_ind.astype(_i32), (0, EP - E),
                  constant_values=PAD_NODE).reshape(NW, NCHUNK, 128)
    row = jnp.pad(row_idx.astype(_i32), (0, EP - E),
                  constant_values=PAD_NODE).reshape(NW, NCHUNK, 128)
    vls = jnp.pad(vals.astype(_f32), (0, EP - E)).reshape(NW, NCHUNK, 128)
    zeros = jnp.zeros((NP,), _f32)

    deg_p, pg_p, fq_p, agg_p, csr_p = _k1(
        pred_p, feats_p, src, dst, col, vls, row, zeros)
    s1_p, w_arr, degc, draw = _k2(deg_p, pg_p, fq_p, src, dst, zeros)
    s2_p, c1 = _k3(s1_p, degc, src, dst, zeros)
    (out16,) = _k4(w_arr, degc, draw, c1, s2_p, csr_p, agg_p, pred_p, b_p)
    return out16[0]
